# baseline (speedup 1.0000x reference)
import jax
import jax.numpy as jnp
from jax import lax
from jax.experimental import pallas as pl
from jax.experimental.pallas import tpu as pltpu
from jax.experimental.pallas import tpu_sc as plsc

D_MODEL = 768
MAX_SEQ_LEN = 2048
BATCH = 64

_INFO = plsc.get_sparse_core_info()
_NC = _INFO.num_cores
_NS = _INFO.num_subcores
_HALF = MAX_SEQ_LEN // _NC
_BPS = BATCH // _NS
_CH = 256
_NCHUNK = _HALF // _CH
_STG = _HALF // _NS


def _body(len_hbm, table_hbm, out_hbm,
          len_v, idx_v, rows_v, sp_tab, sp_zero, gsem, osem):
    cid = lax.axis_index("c")
    sid = lax.axis_index("s")

    for t in range(_STG // 16):
        idx_v[pl.ds(t * 16, 16)] = (lax.iota(jnp.int32, 16)
                                    + (cid * _HALF + sid * _STG + 1 + t * 16))
    pltpu.async_copy(table_hbm.at[idx_v], rows_v, gsem).wait()
    pltpu.sync_copy(rows_v, sp_tab.at[pl.ds(sid * _STG, _STG)])

    @pl.when(sid == 1)
    def _():
        for t in range(_STG // 16):
            idx_v[pl.ds(t * 16, 16)] = jnp.zeros((16,), jnp.int32)
        pltpu.async_copy(table_hbm.at[idx_v], rows_v, gsem).wait()
        for z in range(_CH // _STG):
            pltpu.sync_copy(rows_v, sp_zero.at[pl.ds(z * _STG, _STG)])

    pltpu.sync_copy(len_hbm.at[pl.ds(sid * _BPS * 16, _BPS * 16)], len_v)
    plsc.subcore_barrier()

    nbnd = jnp.int32(0)
    for k in range(_BPS):
        b = sid * _BPS + k
        len_splat = len_v[pl.ds(k * 16, 16)]
        len_s = len_splat[0]
        for c in range(_NCHUNK):
            l0 = cid * _HALF + c * _CH
            dst = out_hbm.at[pl.ds(b * MAX_SEQ_LEN + l0, _CH)]
            is_full = (l0 + _CH) <= len_s
            is_zero = l0 >= len_s
            is_bnd = jnp.logical_not(jnp.logical_or(is_full, is_zero))

            @pl.when(is_full)
            def _():
                pltpu.async_copy(sp_tab.at[pl.ds(c * _CH, _CH)], dst, osem)

            @pl.when(is_zero)
            def _():
                pltpu.async_copy(sp_zero, dst, osem)

            @pl.when(is_bnd)
            def _():
                for h in range(_CH // _STG):
                    for t in range(_STG // 16):
                        pos = (lax.iota(jnp.int32, 16)
                               + (l0 + h * _STG + t * 16 + 1))
                        idx = jnp.where(pos <= len_splat, pos, 0)
                        idx_v[pl.ds(t * 16, 16)] = idx
                    pltpu.async_copy(table_hbm.at[idx_v], rows_v, gsem).wait()
                    pltpu.sync_copy(
                        rows_v,
                        out_hbm.at[pl.ds(b * MAX_SEQ_LEN + l0 + h * _STG, _STG)])

            nbnd = nbnd + is_bnd.astype(jnp.int32)

    def drain(i, carry):
        pltpu.make_async_copy(table_hbm.at[pl.ds(0, _CH)],
                              out_hbm.at[pl.ds(0, _CH)], osem).wait()
        return carry

    lax.fori_loop(0, _BPS * _NCHUNK - nbnd, drain, 0)


def kernel(input_len, pe_table):
    out = pl.kernel(
        _body,
        out_type=jax.ShapeDtypeStruct((BATCH * MAX_SEQ_LEN, D_MODEL), jnp.float32),
        mesh=plsc.VectorSubcoreMesh(core_axis_name="c", subcore_axis_name="s"),
        scratch_types=[
            pltpu.VMEM((_BPS * 16,), jnp.int32),
            pltpu.VMEM((_STG,), jnp.int32),
            pltpu.VMEM((_STG, D_MODEL), jnp.float32),
            pltpu.VMEM_SHARED((_HALF, D_MODEL), jnp.float32),
            pltpu.VMEM_SHARED((_CH, D_MODEL), jnp.float32),
            pltpu.SemaphoreType.DMA,
            pltpu.SemaphoreType.DMA,
        ],
    )(jnp.broadcast_to(input_len.astype(jnp.int32)[:, None],
                       (BATCH, 16)).reshape(BATCH * 16),
      pe_table)
    return out.reshape(BATCH, MAX_SEQ_LEN, D_MODEL)


# speedup vs baseline: 1.8240x; 1.8240x over previous
import jax
import jax.numpy as jnp
from jax import lax
from jax.experimental import pallas as pl
from jax.experimental.pallas import tpu as pltpu
from jax.experimental.pallas import tpu_sc as plsc

D_MODEL = 768
MAX_SEQ_LEN = 2048
BATCH = 64

_INFO = plsc.get_sparse_core_info()
_NC = _INFO.num_cores
_NS = _INFO.num_subcores
_HALF = MAX_SEQ_LEN // _NC
_BPS = BATCH // _NS
_CH = 64
_NCHUNK = _HALF // _CH
_STG = _HALF // _NS


def _body(len_hbm, table_hbm, out_hbm,
          len_v, idx_v, rows_v, sp_tab, sp_zero, gsem, osem):
    cid = lax.axis_index("c")
    sid = lax.axis_index("s")

    for t in range(_STG // 16):
        idx_v[pl.ds(t * 16, 16)] = (lax.iota(jnp.int32, 16)
                                    + (cid * _HALF + sid * _STG + 1 + t * 16))
    pltpu.async_copy(table_hbm.at[idx_v], rows_v, gsem).wait()
    pltpu.sync_copy(rows_v, sp_tab.at[pl.ds(sid * _STG, _STG)])

    @pl.when(sid == 1)
    def _():
        for t in range(_STG // 16):
            idx_v[pl.ds(t * 16, 16)] = jnp.zeros((16,), jnp.int32)
        pltpu.async_copy(table_hbm.at[idx_v], rows_v, gsem).wait()
        for z in range(_CH // _STG):
            pltpu.sync_copy(rows_v, sp_zero.at[pl.ds(z * _STG, _STG)])

    pltpu.sync_copy(len_hbm.at[pl.ds(sid * _BPS * 16, _BPS * 16)], len_v)
    plsc.subcore_barrier()

    nbnd = jnp.int32(0)
    for k in range(_BPS):
        b = sid * _BPS + k
        len_splat = len_v[pl.ds(k * 16, 16)]
        len_s = len_splat[0]
        for c in range(_NCHUNK):
            l0 = cid * _HALF + c * _CH
            dst = out_hbm.at[pl.ds(b * MAX_SEQ_LEN + l0, _CH)]
            is_full = (l0 + _CH) <= len_s
            is_zero = l0 >= len_s
            is_bnd = jnp.logical_not(jnp.logical_or(is_full, is_zero))

            @pl.when(is_full)
            def _():
                pltpu.async_copy(sp_tab.at[pl.ds(c * _CH, _CH)], dst, osem)

            @pl.when(is_zero)
            def _():
                pltpu.async_copy(sp_zero, dst, osem)

            @pl.when(is_bnd)
            def _():
                for h in range(_CH // _STG):
                    for t in range(_STG // 16):
                        pos = (lax.iota(jnp.int32, 16)
                               + (l0 + h * _STG + t * 16 + 1))
                        idx = jnp.where(pos <= len_splat, pos, 0)
                        idx_v[pl.ds(t * 16, 16)] = idx
                    pltpu.async_copy(table_hbm.at[idx_v], rows_v, gsem).wait()
                    pltpu.sync_copy(
                        rows_v,
                        out_hbm.at[pl.ds(b * MAX_SEQ_LEN + l0 + h * _STG, _STG)])

            nbnd = nbnd + is_bnd.astype(jnp.int32)

    def drain(i, carry):
        pltpu.make_async_copy(table_hbm.at[pl.ds(0, _CH)],
                              out_hbm.at[pl.ds(0, _CH)], osem).wait()
        return carry

    lax.fori_loop(0, _BPS * _NCHUNK - nbnd, drain, 0)


def kernel(input_len, pe_table):
    out = pl.kernel(
        _body,
        out_type=jax.ShapeDtypeStruct((BATCH * MAX_SEQ_LEN, D_MODEL), jnp.float32),
        mesh=plsc.VectorSubcoreMesh(core_axis_name="c", subcore_axis_name="s"),
        scratch_types=[
            pltpu.VMEM((_BPS * 16,), jnp.int32),
            pltpu.VMEM((_STG,), jnp.int32),
            pltpu.VMEM((_STG, D_MODEL), jnp.float32),
            pltpu.VMEM_SHARED((_HALF, D_MODEL), jnp.float32),
            pltpu.VMEM_SHARED((_CH, D_MODEL), jnp.float32),
            pltpu.SemaphoreType.DMA,
            pltpu.SemaphoreType.DMA,
        ],
    )(jnp.broadcast_to(input_len.astype(jnp.int32)[:, None],
                       (BATCH, 16)).reshape(BATCH * 16),
      pe_table)
    return out.reshape(BATCH, MAX_SEQ_LEN, D_MODEL)


# speedup vs baseline: 1.8251x; 1.0006x over previous
import jax
import jax.numpy as jnp
from jax import lax
from jax.experimental import pallas as pl
from jax.experimental.pallas import tpu as pltpu
from jax.experimental.pallas import tpu_sc as plsc

D_MODEL = 768
MAX_SEQ_LEN = 2048
BATCH = 64

_INFO = plsc.get_sparse_core_info()
_NC = _INFO.num_cores
_NS = _INFO.num_subcores
_HALF = MAX_SEQ_LEN // _NC
_BPS = BATCH // _NS
_CH = 64
_NCHUNK = _HALF // _CH
_STG = _HALF // _NS
_BSUB = min(_CH, _STG)


def _body(len_hbm, table_hbm, out_hbm,
          len_v, idx_v, rows_v, sp_tab, sp_zero, gsem, osem):
    cid = lax.axis_index("c")
    sid = lax.axis_index("s")

    for g in range(_STG // _BSUB):
        for t in range(_BSUB // 16):
            idx_v[pl.ds(t * 16, 16)] = (
                lax.iota(jnp.int32, 16)
                + (cid * _HALF + sid * _STG + g * _BSUB + 1 + t * 16))
        pltpu.async_copy(table_hbm.at[idx_v], rows_v, gsem).wait()
        pltpu.sync_copy(rows_v,
                        sp_tab.at[pl.ds(sid * _STG + g * _BSUB, _BSUB)])

    @pl.when(sid == 1)
    def _():
        for t in range(_BSUB // 16):
            idx_v[pl.ds(t * 16, 16)] = jnp.zeros((16,), jnp.int32)
        pltpu.async_copy(table_hbm.at[idx_v], rows_v, gsem).wait()
        for z in range(max(1, _CH // _BSUB)):
            pltpu.sync_copy(rows_v, sp_zero.at[pl.ds(z * _BSUB, _BSUB)])

    pltpu.sync_copy(len_hbm.at[pl.ds(sid * _BPS * 16, _BPS * 16)], len_v)
    plsc.subcore_barrier()

    nbnd = jnp.int32(0)
    for k in range(_BPS):
        b = sid * _BPS + k
        len_splat = len_v[pl.ds(k * 16, 16)]
        len_s = len_splat[0]
        for c in range(_NCHUNK):
            l0 = cid * _HALF + c * _CH
            dst = out_hbm.at[pl.ds(b * MAX_SEQ_LEN + l0, _CH)]
            is_full = (l0 + _CH) <= len_s
            is_zero = l0 >= len_s
            is_bnd = jnp.logical_not(jnp.logical_or(is_full, is_zero))

            @pl.when(is_full)
            def _():
                pltpu.async_copy(sp_tab.at[pl.ds(c * _CH, _CH)], dst, osem)

            @pl.when(is_zero)
            def _():
                pltpu.async_copy(sp_zero, dst, osem)

            @pl.when(is_bnd)
            def _():
                for h in range(_CH // _BSUB):
                    for t in range(_BSUB // 16):
                        pos = (lax.iota(jnp.int32, 16)
                               + (l0 + h * _BSUB + t * 16 + 1))
                        idx = jnp.where(pos <= len_splat, pos, 0)
                        idx_v[pl.ds(t * 16, 16)] = idx
                    pltpu.async_copy(table_hbm.at[idx_v], rows_v, gsem).wait()
                    pltpu.sync_copy(
                        rows_v,
                        out_hbm.at[pl.ds(b * MAX_SEQ_LEN + l0 + h * _BSUB,
                                         _BSUB)])

            nbnd = nbnd + is_bnd.astype(jnp.int32)

    def drain(i, carry):
        pltpu.make_async_copy(table_hbm.at[pl.ds(0, _CH)],
                              out_hbm.at[pl.ds(0, _CH)], osem).wait()
        return carry

    lax.fori_loop(0, _BPS * _NCHUNK - nbnd, drain, 0)


def kernel(input_len, pe_table):
    out = pl.kernel(
        _body,
        out_type=jax.ShapeDtypeStruct((BATCH * MAX_SEQ_LEN, D_MODEL), jnp.float32),
        mesh=plsc.VectorSubcoreMesh(core_axis_name="c", subcore_axis_name="s"),
        scratch_types=[
            pltpu.VMEM((_BPS * 16,), jnp.int32),
            pltpu.VMEM((_BSUB,), jnp.int32),
            pltpu.VMEM((_BSUB, D_MODEL), jnp.float32),
            pltpu.VMEM_SHARED((_HALF, D_MODEL), jnp.float32),
            pltpu.VMEM_SHARED((_CH, D_MODEL), jnp.float32),
            pltpu.SemaphoreType.DMA,
            pltpu.SemaphoreType.DMA,
        ],
    )(jnp.broadcast_to(input_len.astype(jnp.int32)[:, None],
                       (BATCH, 16)).reshape(BATCH * 16),
      pe_table)
    return out.reshape(BATCH, MAX_SEQ_LEN, D_MODEL)


# speedup vs baseline: 2.5933x; 1.4209x over previous
import jax
import jax.numpy as jnp
from jax import lax
from jax.experimental import pallas as pl
from jax.experimental.pallas import tpu as pltpu

D_MODEL = 768
MAX_SEQ_LEN = 2048
BATCH = 64
_T = 256


def _tc_body(len_sref, pe_ref, out_ref):
    i = pl.program_id(0)
    j = pl.program_id(1)
    rows = lax.broadcasted_iota(jnp.int32, (_T, D_MODEL), 0) + i * _T + 1
    mask = rows <= len_sref[j]
    out_ref[0] = jnp.where(mask, pe_ref[...], 0.0)


def kernel(input_len, pe_table):
    pe_sub = lax.slice(pe_table, (1, 0), (MAX_SEQ_LEN + 1, D_MODEL))
    return pl.pallas_call(
        _tc_body,
        grid_spec=pltpu.PrefetchScalarGridSpec(
            num_scalar_prefetch=1,
            grid=(MAX_SEQ_LEN // _T, BATCH),
            in_specs=[
                pl.BlockSpec((_T, D_MODEL), lambda i, j, lens: (i, 0)),
            ],
            out_specs=pl.BlockSpec((1, _T, D_MODEL), lambda i, j, lens: (j, i, 0)),
        ),
        out_shape=jax.ShapeDtypeStruct((BATCH, MAX_SEQ_LEN, D_MODEL), jnp.float32),
    )(input_len.astype(jnp.int32), pe_sub)
